# baseline (speedup 1.0000x reference)
import functools

import jax
import jax.numpy as jnp
from jax import lax
from jax.experimental import pallas as pl
from jax.experimental.pallas import tpu as pltpu
from jax.experimental.pallas import tpu_sc as plsc

VOCAB = 1000000
EMBED_DIM = 32
BATCH = 16384
N_FIELDS = 26

_L = 16

_NC = 2
_NS = 16
_NW = _NC * _NS
_BPW = BATCH // _NW
_CHUNK = 32
_NCHUNK = _BPW // _CHUNK
_ROWS = _CHUNK * N_FIELDS


def _shuffle(x, perm):
    dnums = lax.GatherDimensionNumbers(
        offset_dims=(), collapsed_slice_dims=(0,), start_index_map=(0,))
    return lax.gather(x, perm[:, None], dnums, slice_sizes=(1,),
                      mode=lax.GatherScatterMode.PROMISE_IN_BOUNDS)


def _allreduce_sum(x, lane):
    for k in (8, 4, 2, 1):
        perm = jnp.bitwise_xor(lane, jnp.full((_L,), k, jnp.int32))
        x = x + _shuffle(x, perm)
    return x


def _rsqrt_newton(x):
    i = lax.bitcast_convert_type(x, jnp.int32)
    i = jnp.full((_L,), 0x5F3759DF, jnp.int32) - lax.shift_right_logical(
        i, jnp.full((_L,), 1, jnp.int32))
    y = lax.bitcast_convert_type(i, jnp.float32)
    half = 0.5 * x
    for _ in range(3):
        y = y * (1.5 - half * y * y)
    return y


def _pnn_body(idx_hbm, w0_hbm, w1_hbm, v_hbm, out_hbm,
              idx_v, rows_a, rows_b, w1_a, w1_b, out_v, w0_v,
              sem_a, sem_b):
    sid = lax.axis_index("s")
    wid = sid * _NC + lax.axis_index("c")
    base_elt = wid * _BPW
    base_idx = base_elt * N_FIELDS

    pltpu.sync_copy(idx_hbm.at[pl.ds(base_idx, _BPW * N_FIELDS)], idx_v)
    pltpu.sync_copy(w0_hbm, w0_v)

    w0vec = w0_v[...]
    lane = lax.iota(jnp.int32, _L)
    mask10 = jnp.where(lane < (N_FIELDS - _L), 1.0, 0.0).astype(jnp.float32)
    lane2 = lane * 2
    eps = jnp.full((_L,), 1e-8, jnp.float32)

    rows = (rows_a, rows_b)
    w1s = (w1_a, w1_b)
    sems = (sem_a, sem_b)

    _NSPLIT = 4
    _SUB = _ROWS // _NSPLIT

    def issue(c, b):
        for k in range(_NSPLIT):
            sub = idx_v.at[pl.ds(c * _ROWS + k * _SUB, _SUB)]
            pltpu.async_copy(
                v_hbm.at[sub], rows[b].at[pl.ds(k * _SUB, _SUB)], sems[b])
        idxs = idx_v.at[pl.ds(c * _ROWS, _ROWS)]
        pltpu.async_copy(w1_hbm.at[idxs], w1s[b].at[pl.ds(0, _ROWS)], sems[b])

    def drain(c, b):
        for k in range(_NSPLIT):
            sub = idx_v.at[pl.ds(c * _ROWS + k * _SUB, _SUB)]
            pltpu.make_async_copy(
                v_hbm.at[sub], rows[b].at[pl.ds(k * _SUB, _SUB)],
                sems[b]).wait()
        idxs = idx_v.at[pl.ds(c * _ROWS, _ROWS)]
        pltpu.make_async_copy(
            w1_hbm.at[idxs], w1s[b].at[pl.ds(0, _ROWS)], sems[b]).wait()

    def compute_chunk(c, rows_v, w1_v):
        def elt_step(e, _):
            r0 = e * N_FIELDS
            acc = [jnp.zeros((_L,), jnp.float32) for _ in range(8)]
            for f in range(N_FIELDS):
                r = rows_v[r0 + f, pl.ds(0, 2 * _L)]
                v0, v1 = plsc.unpack(r, format=plsc.PackFormat.INTERLEAVED)
                p = 4 * (f & 1)
                acc[p + 0] = acc[p + 0] + v0
                acc[p + 1] = acc[p + 1] + v1
                acc[p + 2] = acc[p + 2] + v0 * v0
                acc[p + 3] = acc[p + 3] + v1 * v1
            s0 = acc[0] + acc[4]
            s1 = acc[1] + acc[5]
            q0 = acc[2] + acc[6]
            q1 = acc[3] + acc[7]

            l0 = w1_v[pl.ds(r0, _L)]
            l1 = w1_v[pl.ds(r0 + _L, _L)] * mask10
            linv = _allreduce_sum(l0 + l1, lane) + w0vec

            fm0 = linv + 0.5 * (s0 * s0 - q0)
            fm1 = linv + 0.5 * (s1 * s1 - q1)

            esv = _allreduce_sum(s0 * s0 + s1 * s1, lane) + eps
            fsv = _allreduce_sum(fm0 * fm0 + fm1 * fm1, lane) + eps
            scale = esv * _rsqrt_newton(esv) * _rsqrt_newton(fsv)

            erow = lax.broadcast_in_dim(e, (_L,), ())
            plsc.store_scatter(out_v, [erow, lane2], fm0 * scale)
            plsc.store_scatter(out_v, [erow, lane2 + 1], fm1 * scale)
            plsc.store_scatter(out_v, [erow, lane2 + 2 * _L], s0)
            plsc.store_scatter(out_v, [erow, lane2 + 2 * _L + 1], s1)
            return ()

        lax.fori_loop(0, _CHUNK, elt_step, (), unroll=4)

        pltpu.sync_copy(
            out_v, out_hbm.at[pl.ds(base_elt + c * _CHUNK, _CHUNK)])

    issue(0, 0)

    def pair_step(i, _):
        for b in range(2):
            c = i * 2 + b

            @pl.when(c + 1 < _NCHUNK)
            def _():
                issue(c + 1, 1 - b)

            drain(c, b)
            compute_chunk(c, rows[b], w1s[b])
        return ()

    lax.fori_loop(0, _NCHUNK // 2, pair_step, (), unroll=False)


@jax.jit
def kernel(input, W0, W1_table, V_table):
    idx_flat = input.reshape(BATCH * N_FIELDS)
    w0_splat = jnp.broadcast_to(W0, (_L,))
    w1_flat = W1_table.reshape(VOCAB)
    v_bf = V_table.astype(jnp.bfloat16)

    mesh = plsc.VectorSubcoreMesh(core_axis_name="c", subcore_axis_name="s")
    run = pl.kernel(
        _pnn_body,
        mesh=mesh,
        compiler_params=pltpu.CompilerParams(
            use_tc_tiling_on_sc=False, needs_layout_passes=False),
        out_type=jax.ShapeDtypeStruct((BATCH, 2 * EMBED_DIM), jnp.float32),
        scratch_types=[
            pltpu.VMEM((_BPW * N_FIELDS,), jnp.int32),
            pltpu.VMEM((_ROWS, EMBED_DIM), jnp.bfloat16),
            pltpu.VMEM((_ROWS, EMBED_DIM), jnp.bfloat16),
            pltpu.VMEM((_ROWS + _L,), jnp.float32),
            pltpu.VMEM((_ROWS + _L,), jnp.float32),
            pltpu.VMEM((_CHUNK, 2 * EMBED_DIM), jnp.float32),
            pltpu.VMEM((_L,), jnp.float32),
            pltpu.SemaphoreType.DMA,
            pltpu.SemaphoreType.DMA,
        ],
    )
    return run(idx_flat, w0_splat, w1_flat, v_bf)


# speedup vs baseline: 1.0991x; 1.0991x over previous
import jax
import jax.numpy as jnp
from jax import lax
from jax.experimental import pallas as pl
from jax.experimental.pallas import tpu as pltpu
from jax.experimental.pallas import tpu_sc as plsc

VOCAB = 1000000
EMBED_DIM = 32
BATCH = 16384
N_FIELDS = 26

_L = 16

_NC = 2
_NS = 16
_NW = _NC * _NS
_BPW = BATCH // _NW
_CHUNK = 8
_NCHUNK = _BPW // _CHUNK
_ROWS = _CHUNK * N_FIELDS
_SUB = _ROWS // 2
_VROWS = VOCAB // 4


def _shuffle(x, perm):
    dnums = lax.GatherDimensionNumbers(
        offset_dims=(), collapsed_slice_dims=(0,), start_index_map=(0,))
    return lax.gather(x, perm[:, None], dnums, slice_sizes=(1,),
                      mode=lax.GatherScatterMode.PROMISE_IN_BOUNDS)


def _allreduce_sum(x, lane):
    for k in (8, 4, 2, 1):
        perm = jnp.bitwise_xor(lane, jnp.full((_L,), k, jnp.int32))
        x = x + _shuffle(x, perm)
    return x


def _rsqrt_newton(x):
    i = lax.bitcast_convert_type(x, jnp.int32)
    i = jnp.full((_L,), 0x5F3759DF, jnp.int32) - lax.shift_right_logical(
        i, jnp.full((_L,), 1, jnp.int32))
    y = lax.bitcast_convert_type(i, jnp.float32)
    half = 0.5 * x
    for _ in range(3):
        y = y * (1.5 - half * y * y)
    return y


def _pnn_body(idx_hbm, idx4_hbm, w0_hbm, w1_hbm, v_hbm, out_hbm,
              idx_v, idx4_v, rows_a, rows_b, w1_a, w1_b, out_v, w0_v,
              sem_a, sem_b):
    wid = lax.axis_index("s") * _NC + lax.axis_index("c")
    base_elt = wid * _BPW
    base_idx = base_elt * N_FIELDS

    pltpu.sync_copy(idx_hbm.at[pl.ds(base_idx, _BPW * N_FIELDS)],
                    idx_v.at[pl.ds(0, _BPW * N_FIELDS)])
    pltpu.sync_copy(idx4_hbm.at[pl.ds(base_idx, _BPW * N_FIELDS)], idx4_v)
    pltpu.sync_copy(w0_hbm, w0_v)

    w0vec = w0_v[...]
    lane = lax.iota(jnp.int32, _L)
    mask10 = jnp.where(lane < (N_FIELDS - _L), 1.0, 0.0).astype(jnp.float32)
    eps = jnp.full((_L,), 1e-8, jnp.float32)

    rows = (rows_a, rows_b)
    w1s = (w1_a, w1_b)
    sems = (sem_a, sem_b)

    def issue(c, b):
        for k in range(2):
            sub4 = idx4_v.at[pl.ds(c * _ROWS + k * _SUB, _SUB)]
            pltpu.async_copy(
                v_hbm.at[sub4], rows[b].at[pl.ds(k * _SUB, _SUB)], sems[b])
            sub = idx_v.at[pl.ds(c * _ROWS + k * _SUB, _SUB)]
            pltpu.async_copy(
                w1_hbm.at[sub], w1s[b].at[pl.ds(k * _SUB, _SUB)], sems[b])

    def drain(c, b):
        for k in range(2):
            sub4 = idx4_v.at[pl.ds(c * _ROWS + k * _SUB, _SUB)]
            pltpu.make_async_copy(
                v_hbm.at[sub4], rows[b].at[pl.ds(k * _SUB, _SUB)],
                sems[b]).wait()
            sub = idx_v.at[pl.ds(c * _ROWS + k * _SUB, _SUB)]
            pltpu.make_async_copy(
                w1_hbm.at[sub], w1s[b].at[pl.ds(k * _SUB, _SUB)],
                sems[b]).wait()

    def compute_chunk(c, rows_v, w1_v):
        def elt_step(e, _):
            r0 = e * N_FIELDS
            iv0 = idx_v[pl.ds(c * _ROWS + r0, _L)]
            iv1 = idx_v[pl.ds(c * _ROWS + r0 + _L, _L)]
            q16 = [(iv0[f] & 3) * 32 for f in range(_L)]
            q10 = [(iv1[f] & 3) * 32 for f in range(N_FIELDS - _L)]
            qoff = q16 + q10

            acc = [jnp.zeros((_L,), jnp.float32) for _ in range(8)]
            for f in range(N_FIELDS):
                v0 = rows_v[r0 + f, pl.ds(qoff[f], _L)]
                v1 = rows_v[r0 + f, pl.ds(qoff[f] + _L, _L)]
                p = 4 * (f & 1)
                acc[p + 0] = acc[p + 0] + v0
                acc[p + 1] = acc[p + 1] + v1
                acc[p + 2] = acc[p + 2] + v0 * v0
                acc[p + 3] = acc[p + 3] + v1 * v1
            s0 = acc[0] + acc[4]
            s1 = acc[1] + acc[5]
            q0 = acc[2] + acc[6]
            q1 = acc[3] + acc[7]

            l0 = w1_v[pl.ds(r0, _L)]
            l1 = w1_v[pl.ds(r0 + _L, _L)] * mask10
            linv = _allreduce_sum(l0 + l1, lane) + w0vec

            fm0 = linv + 0.5 * (s0 * s0 - q0)
            fm1 = linv + 0.5 * (s1 * s1 - q1)

            esv = _allreduce_sum(s0 * s0 + s1 * s1, lane) + eps
            fsv = _allreduce_sum(fm0 * fm0 + fm1 * fm1, lane) + eps
            scale = esv * _rsqrt_newton(esv) * _rsqrt_newton(fsv)

            out_v[e, pl.ds(0, _L)] = fm0 * scale
            out_v[e, pl.ds(_L, _L)] = fm1 * scale
            out_v[e, pl.ds(2 * _L, _L)] = s0
            out_v[e, pl.ds(3 * _L, _L)] = s1
            return ()

        lax.fori_loop(0, _CHUNK, elt_step, (), unroll=4)

        pltpu.sync_copy(
            out_v, out_hbm.at[pl.ds(base_elt + c * _CHUNK, _CHUNK)])

    issue(0, 0)

    def pair_step(i, _):
        for b in range(2):
            c = i * 2 + b

            @pl.when(c + 1 < _NCHUNK)
            def _():
                issue(c + 1, 1 - b)

            drain(c, b)
            compute_chunk(c, rows[b], w1s[b])
        return ()

    lax.fori_loop(0, _NCHUNK // 2, pair_step, (), unroll=False)


@jax.jit
def kernel(input, W0, W1_table, V_table):
    idx_flat = input.reshape(BATCH * N_FIELDS)
    idx4_flat = lax.shift_right_logical(idx_flat, 2)
    w0_splat = jnp.broadcast_to(W0, (_L,))
    w1_flat = W1_table.reshape(VOCAB)
    v_packed = V_table.reshape(_VROWS, 4 * EMBED_DIM)

    mesh = plsc.VectorSubcoreMesh(core_axis_name="c", subcore_axis_name="s")
    run = pl.kernel(
        _pnn_body,
        mesh=mesh,
        compiler_params=pltpu.CompilerParams(use_tc_tiling_on_sc=True),
        out_type=jax.ShapeDtypeStruct((BATCH, 2 * EMBED_DIM), jnp.float32),
        scratch_types=[
            pltpu.VMEM((_BPW * N_FIELDS + _L,), jnp.int32),
            pltpu.VMEM((_BPW * N_FIELDS,), jnp.int32),
            pltpu.VMEM((_ROWS, 4 * EMBED_DIM), jnp.float32),
            pltpu.VMEM((_ROWS, 4 * EMBED_DIM), jnp.float32),
            pltpu.VMEM((_ROWS + _L,), jnp.float32),
            pltpu.VMEM((_ROWS + _L,), jnp.float32),
            pltpu.VMEM((_CHUNK, 2 * EMBED_DIM), jnp.float32),
            pltpu.VMEM((_L,), jnp.float32),
            pltpu.SemaphoreType.DMA,
            pltpu.SemaphoreType.DMA,
        ],
    )
    return run(idx_flat, idx4_flat, w0_splat, w1_flat, v_packed)


# speedup vs baseline: 1.6395x; 1.4917x over previous
import jax
import jax.numpy as jnp
from jax import lax
from jax.experimental import pallas as pl
from jax.experimental.pallas import tpu as pltpu
from jax.experimental.pallas import tpu_sc as plsc
from jax.experimental.layout import Format, Layout, with_layout_constraint

VOCAB = 1000000
EMBED_DIM = 32
BATCH = 16384
N_FIELDS = 26

_L = 16

_NC = 2
_NS = 16
_NW = _NC * _NS
_BPW = BATCH // _NW
_CHUNK = 32
_NCHUNK = _BPW // _CHUNK
_ROWS = _CHUNK * N_FIELDS


def _shuffle(x, perm):
    dnums = lax.GatherDimensionNumbers(
        offset_dims=(), collapsed_slice_dims=(0,), start_index_map=(0,))
    return lax.gather(x, perm[:, None], dnums, slice_sizes=(1,),
                      mode=lax.GatherScatterMode.PROMISE_IN_BOUNDS)


def _allreduce_sum(x, lane):
    for k in (8, 4, 2, 1):
        perm = jnp.bitwise_xor(lane, jnp.full((_L,), k, jnp.int32))
        x = x + _shuffle(x, perm)
    return x


def _rsqrt_newton(x):
    i = lax.bitcast_convert_type(x, jnp.int32)
    i = jnp.full((_L,), 0x5F3759DF, jnp.int32) - lax.shift_right_logical(
        i, jnp.full((_L,), 1, jnp.int32))
    y = lax.bitcast_convert_type(i, jnp.float32)
    half = 0.5 * x
    for _ in range(3):
        y = y * (1.5 - half * y * y)
    return y


def _pnn_body(idx_hbm, w0_hbm, w1_hbm, v_hbm, out_hbm,
              idx_v, rows_a, rows_b, w1_a, w1_b, out_v, w0_v,
              sem_a, sem_b):
    wid = lax.axis_index("s") * _NC + lax.axis_index("c")
    base_elt = wid * _BPW
    base_idx = base_elt * N_FIELDS

    pltpu.sync_copy(idx_hbm.at[pl.ds(base_idx, _BPW * N_FIELDS)], idx_v)
    pltpu.sync_copy(w0_hbm, w0_v)

    w0vec = w0_v[...]
    lane = lax.iota(jnp.int32, _L)
    mask10 = jnp.where(lane < (N_FIELDS - _L), 1.0, 0.0).astype(jnp.float32)
    eps = jnp.full((_L,), 1e-8, jnp.float32)

    rows = (rows_a, rows_b)
    w1s = (w1_a, w1_b)
    sems = (sem_a, sem_b)

    def issue(c, b):
        idxs = idx_v.at[pl.ds(c * _ROWS, _ROWS)]
        pltpu.async_copy(v_hbm.at[idxs], rows[b], sems[b])
        pltpu.async_copy(w1_hbm.at[idxs], w1s[b].at[pl.ds(0, _ROWS)], sems[b])

    def drain(c, b):
        idxs = idx_v.at[pl.ds(c * _ROWS, _ROWS)]
        pltpu.make_async_copy(v_hbm.at[idxs], rows[b], sems[b]).wait()
        pltpu.make_async_copy(
            w1_hbm.at[idxs], w1s[b].at[pl.ds(0, _ROWS)], sems[b]).wait()

    def compute_chunk(c, rows_v, w1_v):
        def elt_step(e, _):
            r0 = e * N_FIELDS
            acc = [jnp.zeros((_L,), jnp.float32) for _ in range(8)]
            for f in range(N_FIELDS):
                v0 = rows_v[r0 + f, pl.ds(0, _L)]
                v1 = rows_v[r0 + f, pl.ds(_L, _L)]
                p = 4 * (f & 1)
                acc[p + 0] = acc[p + 0] + v0
                acc[p + 1] = acc[p + 1] + v1
                acc[p + 2] = acc[p + 2] + v0 * v0
                acc[p + 3] = acc[p + 3] + v1 * v1
            s0 = acc[0] + acc[4]
            s1 = acc[1] + acc[5]
            q0 = acc[2] + acc[6]
            q1 = acc[3] + acc[7]

            l0 = w1_v[pl.ds(r0, _L)]
            l1 = w1_v[pl.ds(r0 + _L, _L)] * mask10
            linv = _allreduce_sum(l0 + l1, lane) + w0vec

            fm0 = linv + 0.5 * (s0 * s0 - q0)
            fm1 = linv + 0.5 * (s1 * s1 - q1)

            esv = _allreduce_sum(s0 * s0 + s1 * s1, lane) + eps
            fsv = _allreduce_sum(fm0 * fm0 + fm1 * fm1, lane) + eps
            scale = esv * _rsqrt_newton(esv) * _rsqrt_newton(fsv)

            out_v[e, pl.ds(0, _L)] = fm0 * scale
            out_v[e, pl.ds(_L, _L)] = fm1 * scale
            out_v[e, pl.ds(2 * _L, _L)] = s0
            out_v[e, pl.ds(3 * _L, _L)] = s1
            return ()

        lax.fori_loop(0, _CHUNK, elt_step, (), unroll=4)

        pltpu.sync_copy(
            out_v, out_hbm.at[pl.ds(base_elt + c * _CHUNK, _CHUNK)])

    issue(0, 0)

    def pair_step(i, _):
        for b in range(2):
            c = i * 2 + b

            @pl.when(c + 1 < _NCHUNK)
            def _():
                issue(c + 1, 1 - b)

            drain(c, b)
            compute_chunk(c, rows[b], w1s[b])
        return ()

    lax.fori_loop(0, _NCHUNK // 2, pair_step, (), unroll=False)


@jax.jit
def kernel(input, W0, W1_table, V_table):
    idx_flat = input.reshape(BATCH * N_FIELDS)
    w0_splat = jnp.broadcast_to(W0, (_L,))
    w1_flat = W1_table.reshape(VOCAB)
    v_rm = with_layout_constraint(
        V_table, Layout(major_to_minor=(0, 1), tiling=((8, 128),)))

    mesh = plsc.VectorSubcoreMesh(core_axis_name="c", subcore_axis_name="s")
    run = pl.kernel(
        _pnn_body,
        mesh=mesh,
        compiler_params=pltpu.CompilerParams(use_tc_tiling_on_sc=False),
        out_type=jax.ShapeDtypeStruct((BATCH, 2 * EMBED_DIM), jnp.float32),
        scratch_types=[
            pltpu.VMEM((_BPW * N_FIELDS,), jnp.int32),
            pltpu.VMEM((_ROWS, EMBED_DIM), jnp.float32),
            pltpu.VMEM((_ROWS, EMBED_DIM), jnp.float32),
            pltpu.VMEM((_ROWS + _L,), jnp.float32),
            pltpu.VMEM((_ROWS + _L,), jnp.float32),
            pltpu.VMEM((_CHUNK, 2 * EMBED_DIM), jnp.float32),
            pltpu.VMEM((_L,), jnp.float32),
            pltpu.SemaphoreType.DMA,
            pltpu.SemaphoreType.DMA,
        ],
    )
    return run(idx_flat, w0_splat, w1_flat, v_rm)
